# baseline (speedup 1.0000x reference)
import jax
import jax.numpy as jnp
from jax import lax
from jax.experimental import pallas as pl
from jax.experimental.pallas import tpu as pltpu
from jax.experimental.pallas import tpu_sc as plsc

NUM_CORES = 2
NUM_SUBCORES = 16
NUM_WORKERS = NUM_CORES * NUM_SUBCORES
LANES = 16

NUM_ROWS = 1000000
BATCH = 16384
DIM = 64
B_PER_W = BATCH // NUM_WORKERS
CHUNK = 128
N_CHUNKS = B_PER_W // CHUNK
N_SLICES = B_PER_W // LANES
NSLOT = 4


def _body(users_hbm, items_hbm, utab_hbm, itab_hbm, out_hbm,
          idx_u, idx_i, ubuf, vbuf, acc, sems):
    wid = lax.axis_index("s") * NUM_CORES + lax.axis_index("c")
    base = wid * B_PER_W

    for c in range(N_CHUNKS):
        pltpu.sync_copy(users_hbm.at[pl.ds(base + c * CHUNK, CHUNK)], idx_u.at[c])
        pltpu.sync_copy(items_hbm.at[pl.ds(base + c * CHUNK, CHUNK)], idx_i.at[c])

    zeros = jnp.zeros((LANES,), jnp.float32)
    for s in range(N_SLICES):
        acc[pl.ds(s * LANES, LANES)] = zeros

    def fire(d, slot):
        for c in range(N_CHUNKS):
            pltpu.async_copy(
                utab_hbm.at[d].at[idx_u.at[c]],
                ubuf.at[slot, pl.ds(c * CHUNK, CHUNK)], sems[slot])
            pltpu.async_copy(
                itab_hbm.at[d].at[idx_i.at[c]],
                vbuf.at[slot, pl.ds(c * CHUNK, CHUNK)], sems[slot])

    def drain(slot):
        for c in range(N_CHUNKS):
            pltpu.make_async_copy(
                utab_hbm.at[0].at[idx_u.at[c]],
                ubuf.at[slot, pl.ds(c * CHUNK, CHUNK)], sems[slot]).wait()
            pltpu.make_async_copy(
                itab_hbm.at[0].at[idx_i.at[c]],
                vbuf.at[slot, pl.ds(c * CHUNK, CHUNK)], sems[slot]).wait()

    for d0 in range(NSLOT - 1):
        fire(d0, d0)

    def outer(g, _):
        for j in range(NSLOT):
            d = g * NSLOT + j
            drain(j)
            for s in range(N_SLICES):
                sl = pl.ds(s * LANES, LANES)
                acc[sl] = acc[sl] + ubuf[j, sl] * vbuf[j, sl]

            @pl.when(d + NSLOT - 1 < DIM)
            def _prefetch():
                fire(d + NSLOT - 1, (j + NSLOT - 1) % NSLOT)
        return _

    lax.fori_loop(0, DIM // NSLOT, outer, None)

    for s in range(N_SLICES):
        sl = pl.ds(s * LANES, LANES)
        acc[sl] = 1.0 / (1.0 + jnp.exp(-acc[sl]))

    pltpu.sync_copy(acc, out_hbm.at[pl.ds(base, B_PER_W)])


@jax.jit
def _run(users, items, user_table, item_table):
    uflat = user_table.T
    iflat = item_table.T
    mesh = plsc.VectorSubcoreMesh(core_axis_name="c", subcore_axis_name="s")
    return pl.kernel(
        _body,
        out_type=jax.ShapeDtypeStruct((BATCH,), jnp.float32),
        mesh=mesh,
        compiler_params=pltpu.CompilerParams(use_tc_tiling_on_sc=False),
        scratch_types=[
            pltpu.VMEM((N_CHUNKS, CHUNK), jnp.int32),
            pltpu.VMEM((N_CHUNKS, CHUNK), jnp.int32),
            pltpu.VMEM((NSLOT, B_PER_W), jnp.float32),
            pltpu.VMEM((NSLOT, B_PER_W), jnp.float32),
            pltpu.VMEM((B_PER_W,), jnp.float32),
            [pltpu.SemaphoreType.DMA] * NSLOT,
        ],
    )(users, items, uflat, iflat)


def kernel(users, items, user_table, item_table):
    return _run(users, items, user_table, item_table)


# speedup vs baseline: 9.0759x; 9.0759x over previous
import jax
import jax.numpy as jnp
from jax import lax
from jax.experimental import pallas as pl
from jax.experimental.pallas import tpu as pltpu
from jax.experimental.pallas import tpu_sc as plsc

NUM_CORES = 2
NUM_SUBCORES = 16
NUM_WORKERS = NUM_CORES * NUM_SUBCORES
LANES = 16

NUM_ROWS = 1000000
BATCH = 16384
DIM = 64
PAIRW = 2 * DIM
NPAIR = NUM_ROWS // 2
B_PER_W = BATCH // NUM_WORKERS
CHUNK = 128
N_CHUNKS = B_PER_W // CHUNK
KSUB = DIM // LANES


def _body(users_hbm, items_hbm, utab_hbm, itab_hbm, out_hbm,
          idx_u, idx_i, par_u, par_i, u_rows, v_rows, out_v, sem):
    wid = lax.axis_index("s") * NUM_CORES + lax.axis_index("c")
    base = wid * B_PER_W

    for c in range(N_CHUNKS):
        pltpu.sync_copy(users_hbm.at[pl.ds(base + c * CHUNK, CHUNK)], idx_u.at[c])
        pltpu.sync_copy(items_hbm.at[pl.ds(base + c * CHUNK, CHUNK)], idx_i.at[c])
    for c in range(N_CHUNKS):
        for s in range(CHUNK // LANES):
            sl = pl.ds(s * LANES, LANES)
            gsl = pl.ds(c * CHUNK + s * LANES, LANES)
            u = idx_u[c, sl]
            v = idx_i[c, sl]
            par_u[gsl] = (u & 1).astype(jnp.float32)
            par_i[gsl] = (v & 1).astype(jnp.float32)
            idx_u[c, sl] = u >> 1
            idx_i[c, sl] = v >> 1

    lanes = lax.iota(jnp.int32, LANES)
    perms = [lanes ^ step for step in (8, 4, 2, 1)]

    def chunk_step(c, _):
        pltpu.async_copy(utab_hbm.at[idx_u.at[c]], u_rows, sem)
        pltpu.async_copy(itab_hbm.at[idx_i.at[c]], v_rows, sem)
        pltpu.make_async_copy(utab_hbm.at[idx_u.at[0]], u_rows, sem).wait()
        pltpu.make_async_copy(itab_hbm.at[idx_i.at[0]], v_rows, sem).wait()

        def group(g, _):
            gsl = pl.ds(c * CHUNK + g * LANES, LANES)
            pu = par_u[gsl]
            pv = par_i[gsl]
            vec = jnp.zeros((LANES,), jnp.float32)
            for j in range(LANES):
                jidx = jnp.full((LANES,), j, jnp.int32)
                pus = pu.at[jidx].get(mode="promise_in_bounds")
                pvs = pv.at[jidx].get(mode="promise_in_bounds")
                r = g * LANES + j
                acc = jnp.zeros((LANES,), jnp.float32)
                for k in range(KSUB):
                    u0 = u_rows[r, pl.ds(k * LANES, LANES)]
                    u1 = u_rows[r, pl.ds(DIM + k * LANES, LANES)]
                    v0 = v_rows[r, pl.ds(k * LANES, LANES)]
                    v1 = v_rows[r, pl.ds(DIM + k * LANES, LANES)]
                    us = u0 + pus * (u1 - u0)
                    vs = v0 + pvs * (v1 - v0)
                    acc = acc + us * vs
                for p in perms:
                    acc = acc + acc.at[p].get(mode="promise_in_bounds")
                vec = jnp.where(lanes == j, acc, vec)
            out_v[gsl] = 1.0 / (1.0 + jnp.exp(-vec))
            return _

        lax.fori_loop(0, CHUNK // LANES, group, None)
        return _

    lax.fori_loop(0, N_CHUNKS, chunk_step, None)

    pltpu.sync_copy(out_v, out_hbm.at[pl.ds(base, B_PER_W)])


@jax.jit
def _run(users, items, user_table, item_table):
    ut = user_table.reshape(NPAIR, PAIRW)
    it = item_table.reshape(NPAIR, PAIRW)
    mesh = plsc.VectorSubcoreMesh(core_axis_name="c", subcore_axis_name="s")
    return pl.kernel(
        _body,
        out_type=jax.ShapeDtypeStruct((BATCH,), jnp.float32),
        mesh=mesh,
        compiler_params=pltpu.CompilerParams(use_tc_tiling_on_sc=True),
        scratch_types=[
            pltpu.VMEM((N_CHUNKS, CHUNK), jnp.int32),
            pltpu.VMEM((N_CHUNKS, CHUNK), jnp.int32),
            pltpu.VMEM((B_PER_W,), jnp.float32),
            pltpu.VMEM((B_PER_W,), jnp.float32),
            pltpu.VMEM((CHUNK, PAIRW), jnp.float32),
            pltpu.VMEM((CHUNK, PAIRW), jnp.float32),
            pltpu.VMEM((B_PER_W,), jnp.float32),
            pltpu.SemaphoreType.DMA,
        ],
    )(users, items, ut, it)


def kernel(users, items, user_table, item_table):
    return _run(users, items, user_table, item_table)


# speedup vs baseline: 15.9685x; 1.7594x over previous
import jax
import jax.numpy as jnp
from jax import lax
from jax.experimental import pallas as pl
from jax.experimental.pallas import tpu as pltpu
from jax.experimental.pallas import tpu_sc as plsc

NUM_CORES = 2
NUM_SUBCORES = 16
NUM_WORKERS = NUM_CORES * NUM_SUBCORES
LANES = 16

NUM_ROWS = 1000000
BATCH = 16384
DIM = 64
TILE_W = 128
N_COLS = 7813
N_FULL = 7812
C_PER_W = (N_COLS + NUM_WORKERS - 1) // NUM_WORKERS
B_PER_W = BATCH // NUM_WORKERS
NVREG = BATCH // LANES
CAP = BATCH + LANES
RB = 128
STAGE_ROWS = BATCH + LANES


def _prefix_sum(x, lanes):
    for k in (1, 2, 4, 8):
        idx = jnp.maximum(lanes - k, 0)
        sh = x.at[idx].get(mode="promise_in_bounds")
        x = x + jnp.where(lanes >= k, sh, 0)
    return x


def _dot_butterfly(acc, perms):
    for p in perms:
        acc = acc + acc.at[p].get(mode="promise_in_bounds")
    return acc


def _scan_body(users_hbm, items_hbm, utab_hbm, itab_hbm,
               utail_hbm, itail_hbm,
               ustage_hbm, vstage_hbm,
               idxall, lvals, lbs, blk, tailbuf, rowbuf, ridx,
               sem_b0, sem_b1, sem_f):
    wid = lax.axis_index("s") * NUM_CORES + lax.axis_index("c")
    lo_c = wid * C_PER_W
    n_c = jnp.minimum(C_PER_W, N_COLS - lo_c)
    lo_v = lo_c * TILE_W
    hi_v = (lo_c + n_c) * TILE_W

    lanes = lax.iota(jnp.int32, LANES)

    def process_table(src_idx_hbm, tab_hbm, tail_hbm, stage_hbm):
        pltpu.sync_copy(src_idx_hbm, idxall)
        pltpu.sync_copy(tail_hbm, tailbuf)

        def fbody(v, cnt):
            u = idxall[pl.ds(pl.multiple_of(v * LANES, LANES), LANES)]
            b = lanes + v * LANES
            m = (u >= lo_v) & (u < hi_v)
            mi = jnp.where(m, 1, 0).astype(jnp.int32)
            pos = cnt + _prefix_sum(mi, lanes) - 1
            plsc.store_scatter(lvals, [pos], u, mask=m)
            plsc.store_scatter(lbs, [pos], b, mask=m)
            return cnt + plsc.all_reduce_population_count(m)[0]

        cnt = lax.fori_loop(0, NVREG, fbody, jnp.int32(0))
        nv = (cnt + LANES - 1) // LANES

        def fire_blk(c, slot, sem):
            cg = lo_c + c
            off = pl.multiple_of(cg * TILE_W, TILE_W)
            pltpu.async_copy(
                utab_like.at[:, pl.ds(off, TILE_W)], blk.at[slot], sem)

        def wait_blk(slot, sem):
            pltpu.make_async_copy(
                utab_like.at[:, pl.ds(0, TILE_W)], blk.at[slot], sem).wait()

        utab_like = tab_hbm

        def reset_ridx():
            dump = jnp.int32(BATCH) + lanes
            for s in range(RB // LANES):
                ridx[0, pl.ds(s * LANES, LANES)] = dump

        reset_ridx()

        def process_col(cg, get_col, carry):
            cnt16 = carry

            def vbody(v, cnt16):
                u = lvals[pl.ds(pl.multiple_of(v * LANES, LANES), LANES)]
                b = lbs[pl.ds(pl.multiple_of(v * LANES, LANES), LANES)]
                m = ((u >> 7) == cg) & (v * LANES + lanes < cnt)
                li = u & (TILE_W - 1)

                def wcond(st):
                    m_rem, cnt16 = st
                    return plsc.all_reduce_population_count(m_rem)[0] > 0

                def wbody(st):
                    m_rem, cnt16 = st
                    lane = plsc.all_reduce_ffs(m_rem)[0]
                    lsp = jnp.zeros((LANES,), jnp.int32) + lane
                    liS = li.at[lsp].get(mode="promise_in_bounds")[0]
                    bS = b.at[lsp].get(mode="promise_in_bounds")[0]
                    for k in range(DIM // LANES):
                        rowbuf[cnt16, pl.ds(k * LANES, LANES)] = (
                            get_col(k, liS))
                    rslot = cnt16 // LANES
                    rlane = cnt16 % LANES
                    rv = ridx[0, pl.ds(pl.multiple_of(rslot * LANES, LANES),
                                       LANES)]
                    ridx[0, pl.ds(pl.multiple_of(rslot * LANES, LANES),
                                  LANES)] = jnp.where(lanes == rlane, bS, rv)
                    m_rem = m_rem & (lanes != lane)
                    return m_rem, cnt16 + 1

                m_rem, cnt16 = lax.while_loop(wcond, wbody, (m, cnt16))

                @pl.when(cnt16 > RB - LANES)
                def _flv():
                    pltpu.async_copy(rowbuf, stage_hbm.at[ridx.at[0]], sem_f)
                    pltpu.make_async_copy(
                        rowbuf, stage_hbm.at[ridx.at[0]], sem_f).wait()
                    reset_ridx()

                return jnp.where(cnt16 > RB - LANES, 0, cnt16)

            cnt16 = lax.fori_loop(0, nv, vbody, cnt16)

            @pl.when(cnt16 > RB - LANES)
            def _fl():
                pltpu.async_copy(rowbuf, stage_hbm.at[ridx.at[0]], sem_f)
                pltpu.make_async_copy(
                    rowbuf, stage_hbm.at[ridx.at[0]], sem_f).wait()
                reset_ridx()

            cnt16 = jnp.where(cnt16 > RB - LANES, 0, cnt16)
            return cnt16

        def blk_col(slot):
            def get_col(k, liS):
                return plsc.load_gather(
                    blk,
                    [jnp.zeros((LANES,), jnp.int32) + slot,
                     lanes + k * LANES,
                     jnp.zeros((LANES,), jnp.int32) + liS])
            return get_col

        def tail_col(k, liS):
            return plsc.load_gather(
                tailbuf,
                [lanes + k * LANES,
                 jnp.zeros((LANES,), jnp.int32) + liS])

        nf = jnp.minimum(n_c, N_FULL - lo_c)
        fire_blk(0, 0, sem_b0)

        def cpair(cp, cnt16):
            c0 = cp * 2
            c1 = c0 + 1

            @pl.when(c1 < nf)
            def _f1():
                fire_blk(c1, 1, sem_b1)

            @pl.when(c0 < nf)
            def _w0():
                wait_blk(0, sem_b0)

            cnt16 = process_col(lo_c + c0, blk_col(0), cnt16)

            @pl.when(c0 + 2 < nf)
            def _f2():
                fire_blk(c0 + 2, 0, sem_b0)

            @pl.when(c1 < nf)
            def _p1():
                wait_blk(1, sem_b1)

            cnt16 = process_col(lo_c + c1, blk_col(1), cnt16)
            return cnt16

        cnt16 = lax.fori_loop(0, (C_PER_W + 1) // 2, cpair, jnp.int32(0))

        cnt16 = jnp.where(lo_c + n_c >= N_COLS,
                          process_col(N_FULL, tail_col, cnt16), cnt16)

        pltpu.async_copy(rowbuf, stage_hbm.at[ridx.at[0]], sem_f)
        pltpu.make_async_copy(rowbuf, stage_hbm.at[ridx.at[0]], sem_f).wait()

    process_table(users_hbm, utab_hbm, utail_hbm, ustage_hbm)
    process_table(items_hbm, itab_hbm, itail_hbm, vstage_hbm)


def _dot_body(ustage_hbm, vstage_hbm, out_hbm, ub, vb, out_v, sem):
    wid = lax.axis_index("s") * NUM_CORES + lax.axis_index("c")
    base = wid * B_PER_W
    lanes = lax.iota(jnp.int32, LANES)
    perms = [lanes ^ step for step in (8, 4, 2, 1)]

    def chunk(c, _):
        off = pl.multiple_of(base + c * TILE_W, 8)
        pltpu.async_copy(ustage_hbm.at[pl.ds(off, TILE_W), :], ub, sem)
        pltpu.async_copy(vstage_hbm.at[pl.ds(off, TILE_W), :], vb, sem)
        pltpu.make_async_copy(
            ustage_hbm.at[pl.ds(0, TILE_W), :], ub, sem).wait()
        pltpu.make_async_copy(
            vstage_hbm.at[pl.ds(0, TILE_W), :], vb, sem).wait()

        def group(g, _):
            vec = jnp.zeros((LANES,), jnp.float32)
            for j in range(LANES):
                r = g * LANES + j
                acc = ub[r, pl.ds(0, LANES)] * vb[r, pl.ds(0, LANES)]
                for k in range(1, DIM // LANES):
                    acc += (ub[r, pl.ds(k * LANES, LANES)]
                            * vb[r, pl.ds(k * LANES, LANES)])
                acc = _dot_butterfly(acc, perms)
                vec = jnp.where(lanes == j, acc, vec)
            out_v[pl.ds(c * TILE_W + g * LANES, LANES)] = (
                1.0 / (1.0 + jnp.exp(-vec)))
            return _

        lax.fori_loop(0, TILE_W // LANES, group, None)
        return _

    lax.fori_loop(0, B_PER_W // TILE_W, chunk, None)
    pltpu.sync_copy(out_v, out_hbm.at[pl.ds(base, B_PER_W)])


@jax.jit
def _run(users, items, user_table, item_table):
    utT = user_table.T
    itT = item_table.T
    tail_w = NUM_ROWS - N_FULL * TILE_W
    utail = jnp.pad(utT[:, N_FULL * TILE_W:], ((0, 0), (0, TILE_W - tail_w)))
    itail = jnp.pad(itT[:, N_FULL * TILE_W:], ((0, 0), (0, TILE_W - tail_w)))
    mesh = plsc.VectorSubcoreMesh(core_axis_name="c", subcore_axis_name="s")
    ustage, vstage = pl.kernel(
        _scan_body,
        out_type=(jax.ShapeDtypeStruct((STAGE_ROWS, TILE_W), jnp.float32),
                  jax.ShapeDtypeStruct((STAGE_ROWS, TILE_W), jnp.float32)),
        mesh=mesh,
        compiler_params=pltpu.CompilerParams(use_tc_tiling_on_sc=True, needs_layout_passes=False),
        scratch_types=[
            pltpu.VMEM((BATCH,), jnp.int32),
            pltpu.VMEM((CAP,), jnp.int32),
            pltpu.VMEM((CAP,), jnp.int32),
            pltpu.VMEM((2, DIM, TILE_W), jnp.float32),
            pltpu.VMEM((DIM, TILE_W), jnp.float32),
            pltpu.VMEM((RB, TILE_W), jnp.float32),
            pltpu.VMEM((1, RB), jnp.int32),
            pltpu.SemaphoreType.DMA,
            pltpu.SemaphoreType.DMA,
            pltpu.SemaphoreType.DMA,
        ],
    )(users, items, utT, itT, utail, itail)

    return pl.kernel(
        _dot_body,
        out_type=jax.ShapeDtypeStruct((BATCH,), jnp.float32),
        mesh=mesh,
        compiler_params=pltpu.CompilerParams(use_tc_tiling_on_sc=True, needs_layout_passes=False),
        scratch_types=[
            pltpu.VMEM((TILE_W, TILE_W), jnp.float32),
            pltpu.VMEM((TILE_W, TILE_W), jnp.float32),
            pltpu.VMEM((B_PER_W,), jnp.float32),
            pltpu.SemaphoreType.DMA,
        ],
    )(ustage, vstage)


def kernel(users, items, user_table, item_table):
    return _run(users, items, user_table, item_table)


# speedup vs baseline: 25.2281x; 1.5799x over previous
import jax
import jax.numpy as jnp
from jax import lax
from jax.experimental import pallas as pl
from jax.experimental.pallas import tpu as pltpu
from jax.experimental.pallas import tpu_sc as plsc

NUM_CORES = 2
NUM_SUBCORES = 16
NUM_WORKERS = NUM_CORES * NUM_SUBCORES
LANES = 16

NUM_ROWS = 1000000
BATCH = 16384
DIM = 64
TILE_W = 128
SCW = 256
N_SCOLS = (NUM_ROWS + SCW - 1) // SCW
N_SFULL = NUM_ROWS // SCW
TAIL_W = NUM_ROWS - N_SFULL * SCW
C_PER_W = (N_SCOLS + NUM_WORKERS - 1) // NUM_WORKERS
B_PER_W = BATCH // NUM_WORKERS
ICHUNK = 2048
CAP = BATCH + LANES
RB = 64
NSLOT = 4
STAGE_ROWS = BATCH + LANES


def _prefix_sum(x, lanes):
    for k in (1, 2, 4, 8):
        idx = jnp.maximum(lanes - k, 0)
        sh = x.at[idx].get(mode="promise_in_bounds")
        x = x + jnp.where(lanes >= k, sh, 0)
    return x


def _dot_butterfly(acc, perms):
    for p in perms:
        acc = acc + acc.at[p].get(mode="promise_in_bounds")
    return acc


def _scan_body(users_hbm, items_hbm, utab_hbm, itab_hbm,
               utail_hbm, itail_hbm,
               ustage_hbm, vstage_hbm,
               idxall, lvals, lbs, blk, tailbuf, rowbuf, ridx,
               sems):
    wid = lax.axis_index("s") * NUM_CORES + lax.axis_index("c")
    lo_c = wid * C_PER_W
    n_c = jnp.minimum(C_PER_W, N_SCOLS - lo_c)
    lo_v = lo_c * SCW
    hi_v = (lo_c + n_c) * SCW

    lanes = lax.iota(jnp.int32, LANES)

    def process_table(src_idx_hbm, tab_hbm, tail_hbm, stage_hbm, sem_f):
        pltpu.sync_copy(tail_hbm, tailbuf)

        def fchunk(ch, cnt):
            pltpu.sync_copy(
                src_idx_hbm.at[pl.ds(pl.multiple_of(ch * ICHUNK, 8), ICHUNK)],
                idxall)

            def fbody(v, cnt):
                u = idxall[pl.ds(pl.multiple_of(v * LANES, LANES), LANES)]
                b = ch * ICHUNK + v * LANES + lanes
                m = (u >= lo_v) & (u < hi_v)
                mi = jnp.where(m, 1, 0).astype(jnp.int32)
                pos = cnt + _prefix_sum(mi, lanes) - 1
                plsc.store_scatter(lvals, [pos], u, mask=m)
                plsc.store_scatter(lbs, [pos], b, mask=m)
                return cnt + plsc.all_reduce_population_count(m)[0]

            return lax.fori_loop(0, ICHUNK // LANES, fbody, cnt)

        cnt = lax.fori_loop(0, BATCH // ICHUNK, fchunk, jnp.int32(0))
        nv = (cnt + LANES - 1) // LANES

        def fire_blk(c, slot):
            cg = lo_c + c
            off = pl.multiple_of(cg * SCW, SCW)
            pltpu.async_copy(
                tab_hbm.at[:, pl.ds(off, SCW)], blk.at[slot], sems[slot])

        def wait_blk(slot):
            pltpu.make_async_copy(
                tab_hbm.at[:, pl.ds(0, SCW)], blk.at[slot], sems[slot]).wait()

        def reset_ridx():
            dump = jnp.int32(BATCH) + lanes
            for s in range(RB // LANES):
                ridx[0, pl.ds(s * LANES, LANES)] = dump

        reset_ridx()

        def process_col(cg, get_col, carry):
            cnt16 = carry

            def vbody(v, cnt16):
                u = lvals[pl.ds(pl.multiple_of(v * LANES, LANES), LANES)]
                b = lbs[pl.ds(pl.multiple_of(v * LANES, LANES), LANES)]
                m = ((u >> 8) == cg) & (v * LANES + lanes < cnt)
                li = u & (SCW - 1)

                def wcond(st):
                    m_rem, cnt16 = st
                    return plsc.all_reduce_population_count(m_rem)[0] > 0

                def wbody(st):
                    m_rem, cnt16 = st
                    lane = plsc.all_reduce_ffs(m_rem)[0]
                    lsp = jnp.zeros((LANES,), jnp.int32) + lane
                    liS = li.at[lsp].get(mode="promise_in_bounds")[0]
                    bS = b.at[lsp].get(mode="promise_in_bounds")[0]
                    for k in range(DIM // LANES):
                        rowbuf[cnt16, pl.ds(k * LANES, LANES)] = (
                            get_col(k, liS))
                    rslot = cnt16 // LANES
                    rlane = cnt16 % LANES
                    rsl = pl.ds(pl.multiple_of(rslot * LANES, LANES), LANES)
                    ridx[0, rsl] = jnp.where(lanes == rlane, bS, ridx[0, rsl])
                    m_rem = m_rem & (lanes != lane)
                    return m_rem, cnt16 + 1

                m_rem, cnt16 = lax.while_loop(wcond, wbody, (m, cnt16))

                @pl.when(cnt16 > RB - LANES)
                def _flv():
                    pltpu.async_copy(rowbuf, stage_hbm.at[ridx.at[0]], sem_f)
                    pltpu.make_async_copy(
                        rowbuf, stage_hbm.at[ridx.at[0]], sem_f).wait()
                    reset_ridx()

                return jnp.where(cnt16 > RB - LANES, 0, cnt16)

            return lax.fori_loop(0, nv, vbody, cnt16)

        def blk_col(slot):
            def get_col(k, liS):
                return plsc.load_gather(
                    blk,
                    [jnp.zeros((LANES,), jnp.int32) + slot,
                     lanes + k * LANES,
                     jnp.zeros((LANES,), jnp.int32) + liS])
            return get_col

        def tail_col(k, liS):
            return plsc.load_gather(
                tailbuf,
                [lanes + k * LANES,
                 jnp.zeros((LANES,), jnp.int32) + liS])

        nf = jnp.minimum(n_c, N_SFULL - lo_c)
        for c0 in range(NSLOT - 1):
            @pl.when(c0 < nf)
            def _fp():
                fire_blk(c0, c0)

        def cquad(cq, cnt16):
            for j in range(NSLOT):
                c = cq * NSLOT + j

                @pl.when(c < nf)
                def _w():
                    wait_blk(j)

                cnt16 = process_col(lo_c + c, blk_col(j), cnt16)

                @pl.when(c + NSLOT - 1 < nf)
                def _f():
                    fire_blk(c + NSLOT - 1, (j + NSLOT - 1) % NSLOT)

            return cnt16

        cnt16 = lax.fori_loop(0, (C_PER_W + NSLOT - 1) // NSLOT,
                              cquad, jnp.int32(0))

        cnt16 = process_col(N_SFULL, tail_col, cnt16)

        pltpu.async_copy(rowbuf, stage_hbm.at[ridx.at[0]], sem_f)
        pltpu.make_async_copy(rowbuf, stage_hbm.at[ridx.at[0]], sem_f).wait()

    process_table(users_hbm, utab_hbm, utail_hbm, ustage_hbm, sems[NSLOT])
    process_table(items_hbm, itab_hbm, itail_hbm, vstage_hbm, sems[NSLOT + 1])


def _dot_body(ustage_hbm, vstage_hbm, out_hbm, ub, vb, out_v, sem):
    wid = lax.axis_index("s") * NUM_CORES + lax.axis_index("c")
    base = wid * B_PER_W
    lanes = lax.iota(jnp.int32, LANES)
    perms = [lanes ^ step for step in (8, 4, 2, 1)]

    def chunk(c, _):
        off = pl.multiple_of(base + c * TILE_W, 8)
        pltpu.async_copy(ustage_hbm.at[pl.ds(off, TILE_W), :], ub, sem)
        pltpu.async_copy(vstage_hbm.at[pl.ds(off, TILE_W), :], vb, sem)
        pltpu.make_async_copy(
            ustage_hbm.at[pl.ds(0, TILE_W), :], ub, sem).wait()
        pltpu.make_async_copy(
            vstage_hbm.at[pl.ds(0, TILE_W), :], vb, sem).wait()

        def group(g, _):
            vec = jnp.zeros((LANES,), jnp.float32)
            for j in range(LANES):
                r = g * LANES + j
                acc = ub[r, pl.ds(0, LANES)] * vb[r, pl.ds(0, LANES)]
                for k in range(1, DIM // LANES):
                    acc += (ub[r, pl.ds(k * LANES, LANES)]
                            * vb[r, pl.ds(k * LANES, LANES)])
                acc = _dot_butterfly(acc, perms)
                vec = jnp.where(lanes == j, acc, vec)
            out_v[pl.ds(c * TILE_W + g * LANES, LANES)] = (
                1.0 / (1.0 + jnp.exp(-vec)))
            return _

        lax.fori_loop(0, TILE_W // LANES, group, None)
        return _

    lax.fori_loop(0, B_PER_W // TILE_W, chunk, None)
    pltpu.sync_copy(out_v, out_hbm.at[pl.ds(base, B_PER_W)])


@jax.jit
def _run(users, items, user_table, item_table):
    utT = user_table.T
    itT = item_table.T
    utail = jnp.pad(utT[:, N_SFULL * SCW:], ((0, 0), (0, TILE_W - TAIL_W)))
    itail = jnp.pad(itT[:, N_SFULL * SCW:], ((0, 0), (0, TILE_W - TAIL_W)))
    mesh = plsc.VectorSubcoreMesh(core_axis_name="c", subcore_axis_name="s")
    ustage, vstage = pl.kernel(
        _scan_body,
        out_type=(jax.ShapeDtypeStruct((STAGE_ROWS, TILE_W), jnp.float32),
                  jax.ShapeDtypeStruct((STAGE_ROWS, TILE_W), jnp.float32)),
        mesh=mesh,
        compiler_params=pltpu.CompilerParams(
            use_tc_tiling_on_sc=True, needs_layout_passes=False),
        scratch_types=[
            pltpu.VMEM((ICHUNK,), jnp.int32),
            pltpu.VMEM((CAP,), jnp.int32),
            pltpu.VMEM((CAP,), jnp.int32),
            pltpu.VMEM((NSLOT, DIM, SCW), jnp.float32),
            pltpu.VMEM((DIM, TILE_W), jnp.float32),
            pltpu.VMEM((RB, TILE_W), jnp.float32),
            pltpu.VMEM((1, RB), jnp.int32),
            [pltpu.SemaphoreType.DMA] * (NSLOT + 2),
        ],
    )(users, items, utT, itT, utail, itail)

    return pl.kernel(
        _dot_body,
        out_type=jax.ShapeDtypeStruct((BATCH,), jnp.float32),
        mesh=mesh,
        compiler_params=pltpu.CompilerParams(
            use_tc_tiling_on_sc=True, needs_layout_passes=False),
        scratch_types=[
            pltpu.VMEM((TILE_W, TILE_W), jnp.float32),
            pltpu.VMEM((TILE_W, TILE_W), jnp.float32),
            pltpu.VMEM((B_PER_W,), jnp.float32),
            pltpu.SemaphoreType.DMA,
        ],
    )(ustage, vstage)


def kernel(users, items, user_table, item_table):
    return _run(users, items, user_table, item_table)
